# baseline (speedup 1.0000x reference)
import functools

import jax
import jax.numpy as jnp
import numpy as np
from jax import lax
from jax.experimental import pallas as pl
from jax.experimental.pallas import tpu as pltpu
from jax.experimental.pallas import tpu_sc as plsc

N = 10000
E = 320000
D = 128
G = 64
NC, NS = 2, 16
NW = NC * NS
NP = 10240
R = 1024
EB = 96
K = 105
WSLOTS = K * EB
CH = 15
NCH = K // CH
T = K // 3
ACC_R = 10112
ROWS_PER_TILE = ACC_R // NS


def _agg_body(h_hbm, src_hbm, dst_hbm, zeros_hbm, out_hbm,
              acc, src_v, dst_v, r0, r1, r2,
              g0, g1, g2, s0, s1, s2, ixs, ixd):
    cid = lax.axis_index("c")
    sid = lax.axis_index("s")
    row0 = sid * ROWS_PER_TILE
    pltpu.sync_copy(zeros_hbm.at[pl.ds(row0, ROWS_PER_TILE)],
                    acc.at[pl.ds(row0, ROWS_PER_TILE)])
    pltpu.sync_copy(src_hbm.at[cid, sid, 0], src_v.at[0])
    pltpu.sync_copy(dst_hbm.at[cid, sid, 0], dst_v.at[0])
    plsc.subcore_barrier()

    def gather(p, jj, rv, sem):
        pltpu.make_async_copy(h_hbm.at[src_v.at[p, jj]], rv, sem).start()

    def gather_wait(rv, sem):
        pltpu.make_async_copy(h_hbm.at[src_v.at[0, 0]], rv, sem).wait()

    def scatter(p, jj, rv, sem):
        pltpu.make_async_copy(rv, acc.at[dst_v.at[p, jj]], sem).start(add=True)

    def scatter_wait(rv, sem):
        pltpu.make_async_copy(rv, acc.at[dst_v.at[0, 0]], sem).wait()

    gather(0, 0, r0, g0)
    gather(0, 1, r1, g1)

    def step(t, carry):
        m = t // 5
        p = lax.rem(m, 2)
        jj0 = 3 * lax.rem(t, 5)

        @pl.when(t > 0)
        def _():
            scatter_wait(r2, s2)

        @pl.when(jnp.logical_and(lax.rem(t, 5) == 0, t < (NCH - 1) * 5))
        def _():
            pn = lax.rem(m + 1, 2)
            pltpu.make_async_copy(src_hbm.at[cid, sid, m + 1],
                                  src_v.at[pn], ixs).start()
            pltpu.make_async_copy(dst_hbm.at[cid, sid, m + 1],
                                  dst_v.at[pn], ixd).start()

        gather(p, jj0 + 2, r2, g2)
        gather_wait(r0, g0)
        scatter(p, jj0, r0, s0)
        gather_wait(r1, g1)
        scatter(p, jj0 + 1, r1, s1)

        @pl.when(jnp.logical_and(lax.rem(t, 5) == 4, t < (NCH - 1) * 5))
        def _():
            pltpu.make_async_copy(src_hbm.at[cid, sid, 0],
                                  src_v.at[0], ixs).wait()
            pltpu.make_async_copy(dst_hbm.at[cid, sid, 0],
                                  dst_v.at[0], ixd).wait()

        jn0 = 3 * t + 3
        mn = jn0 // CH
        pn0 = lax.rem(mn, 2)
        jjn0 = jn0 - mn * CH
        scatter_wait(r0, s0)

        @pl.when(t < T - 1)
        def _():
            gather(pn0, jjn0, r0, g0)

        jn1 = 3 * t + 4
        mn1 = jn1 // CH
        pn1 = lax.rem(mn1, 2)
        jjn1 = jn1 - mn1 * CH
        scatter_wait(r1, s1)

        @pl.when(t < T - 1)
        def _():
            gather(pn1, jjn1, r1, g1)

        gather_wait(r2, g2)
        scatter(p, jj0 + 2, r2, s2)
        return carry

    lax.fori_loop(0, T, step, 0)
    scatter_wait(r2, s2)
    plsc.subcore_barrier()
    pltpu.sync_copy(acc.at[pl.ds(row0, ROWS_PER_TILE)],
                    out_hbm.at[cid, pl.ds(row0, ROWS_PER_TILE)])


_agg = functools.partial(
    pl.kernel,
    out_type=jax.ShapeDtypeStruct((NC, ACC_R, D), jnp.float32),
    mesh=plsc.VectorSubcoreMesh(core_axis_name="c", subcore_axis_name="s"),
    scratch_types=[
        pltpu.VMEM_SHARED((ACC_R, D), jnp.float32),
        pltpu.VMEM((2, CH, EB), jnp.int32),
        pltpu.VMEM((2, CH, EB), jnp.int32),
        pltpu.VMEM((EB, D), jnp.float32),
        pltpu.VMEM((EB, D), jnp.float32),
        pltpu.VMEM((EB, D), jnp.float32),
        pltpu.SemaphoreType.DMA,
        pltpu.SemaphoreType.DMA,
        pltpu.SemaphoreType.DMA,
        pltpu.SemaphoreType.DMA,
        pltpu.SemaphoreType.DMA,
        pltpu.SemaphoreType.DMA,
        pltpu.SemaphoreType.DMA,
        pltpu.SemaphoreType.DMA,
    ],
)(_agg_body)


def _row_valid(i):
    return (lax.broadcasted_iota(jnp.int32, (R, D), 0) + i * R) < N


def _mlp_body(h_ref, p_ref, w1_ref, b1_ref, w2_ref, b2_ref, o_ref):
    i = pl.program_id(0)
    z = h_ref[...] + p_ref[0] + p_ref[1]
    z = jnp.maximum(
        jnp.dot(z, w1_ref[...], preferred_element_type=jnp.float32) + b1_ref[...],
        0.0)
    z = jnp.dot(z, w2_ref[...], preferred_element_type=jnp.float32) + b2_ref[...]
    o_ref[...] = jnp.where(_row_valid(i), jnp.maximum(z, 0.0), 0.0)


_mlp = pl.pallas_call(
    _mlp_body,
    grid=(NP // R,),
    in_specs=[
        pl.BlockSpec((R, D), lambda i: (i, 0)),
        pl.BlockSpec((NC, R, D), lambda i: (0, i, 0)),
        pl.BlockSpec((D, D), lambda i: (0, 0)),
        pl.BlockSpec((1, D), lambda i: (0, 0)),
        pl.BlockSpec((D, D), lambda i: (0, 0)),
        pl.BlockSpec((1, D), lambda i: (0, 0)),
    ],
    out_specs=pl.BlockSpec((R, D), lambda i: (i, 0)),
    out_shape=jax.ShapeDtypeStruct((NP, D), jnp.float32),
)


def _mlp_pool_body(h_ref, p_ref, batch_ref, w1_ref, b1_ref, w2_ref, b2_ref,
                   wf1_ref, bf1_ref, wf2_ref, bf2_ref, o_ref, pooled):
    i = pl.program_id(0)
    z = h_ref[...] + p_ref[0] + p_ref[1]
    z = jnp.maximum(
        jnp.dot(z, w1_ref[...], preferred_element_type=jnp.float32) + b1_ref[...],
        0.0)
    z = jnp.dot(z, w2_ref[...], preferred_element_type=jnp.float32) + b2_ref[...]
    h3 = jnp.where(_row_valid(i), jnp.maximum(z, 0.0), 0.0)
    seg = batch_ref[0, 0, :]
    m = (lax.broadcasted_iota(jnp.int32, (G, R), 0) == seg[None, :]
         ).astype(jnp.float32)
    part = jnp.dot(m, h3, preferred_element_type=jnp.float32)

    @pl.when(i == 0)
    def _():
        pooled[...] = part

    @pl.when(i > 0)
    def _():
        pooled[...] += part

    @pl.when(i == pl.num_programs(0) - 1)
    def _():
        q = jnp.maximum(
            jnp.dot(pooled[...], wf1_ref[...],
                    preferred_element_type=jnp.float32) + bf1_ref[...],
            0.0)
        o_ref[...] = (jnp.dot(q, wf2_ref[...],
                              preferred_element_type=jnp.float32)
                      + bf2_ref[...])


_mlp_pool = pl.pallas_call(
    _mlp_pool_body,
    grid=(NP // R,),
    in_specs=[
        pl.BlockSpec((R, D), lambda i: (i, 0)),
        pl.BlockSpec((NC, R, D), lambda i: (0, i, 0)),
        pl.BlockSpec((1, 1, R), lambda i: (i, 0, 0)),
        pl.BlockSpec((D, D), lambda i: (0, 0)),
        pl.BlockSpec((1, D), lambda i: (0, 0)),
        pl.BlockSpec((D, D), lambda i: (0, 0)),
        pl.BlockSpec((1, D), lambda i: (0, 0)),
        pl.BlockSpec((D, D), lambda i: (0, 0)),
        pl.BlockSpec((1, D), lambda i: (0, 0)),
        pl.BlockSpec((D, D), lambda i: (0, 0)),
        pl.BlockSpec((1, D), lambda i: (0, 0)),
    ],
    out_specs=pl.BlockSpec((G, D), lambda i: (0, 0)),
    out_shape=jax.ShapeDtypeStruct((G, D), jnp.float32),
    scratch_shapes=[pltpu.VMEM((G, D), jnp.float32)],
)


def kernel(x, edge_index, edge_attr, batch,
           W1_0, b1_0, W2_0, b2_0, W1_1, b1_1, W2_1, b2_1,
           W1_2, b1_2, W2_2, b2_2, Wf1, bf1, Wf2, bf2):
    del edge_attr
    w_ids = np.arange(NW, dtype=np.int32)[:, None]
    pad_i = np.arange(WSLOTS - E // NW, dtype=np.int32)[None, :]
    pad_src = N + (w_ids * 80 + pad_i) % (NP - N)
    pad_dst = (w_ids * 313 + pad_i * 97) % N
    src = jnp.concatenate(
        [edge_index[0].astype(jnp.int32).reshape(NW, E // NW),
         jnp.asarray(pad_src)], axis=1).reshape(NC, NS, NCH, CH, EB)
    dst = jnp.concatenate(
        [edge_index[1].astype(jnp.int32).reshape(NW, E // NW),
         jnp.asarray(pad_dst)], axis=1).reshape(NC, NS, NCH, CH, EB)
    zeros = jnp.zeros((ACC_R, D), jnp.float32)
    h = jnp.concatenate([x, jnp.zeros((NP - N, D), x.dtype)])
    batch_p = jnp.concatenate(
        [batch.astype(jnp.int32),
         jnp.full((NP - N,), G, jnp.int32)]).reshape(NP // R, 1, R)

    parts = _agg(h, src, dst, zeros)
    h = _mlp(h, parts, W1_0, b1_0.reshape(1, D), W2_0, b2_0.reshape(1, D))
    parts = _agg(h, src, dst, zeros)
    h = _mlp(h, parts, W1_1, b1_1.reshape(1, D), W2_1, b2_1.reshape(1, D))
    parts = _agg(h, src, dst, zeros)
    out = _mlp_pool(h, parts, batch_p,
                    W1_2, b1_2.reshape(1, D), W2_2, b2_2.reshape(1, D),
                    Wf1, bf1.reshape(1, D), Wf2, bf2.reshape(1, D))
    return out


# speedup vs baseline: 1.1265x; 1.1265x over previous
import functools

import jax
import jax.numpy as jnp
import numpy as np
from jax import lax
from jax.experimental import pallas as pl
from jax.experimental.pallas import tpu as pltpu
from jax.experimental.pallas import tpu_sc as plsc

N = 10000
E = 320000
D = 128
G = 64
NC, NS = 2, 16
NP = 10240
R = 1024
EB = 128
K = 80
CH = 20
NCH = K // CH
EP = NC * NS * K * EB
ROWS_PER_TILE = NP // NS


def _agg_body(h_hbm, src_hbm, dst_hbm, zeros_hbm, out_hbm,
              acc, src_v, dst_v, r0, r1, g0, g1, s0, s1, ixs, ixd):
    cid = lax.axis_index("c")
    sid = lax.axis_index("s")
    row0 = sid * ROWS_PER_TILE
    pltpu.sync_copy(zeros_hbm.at[pl.ds(row0, ROWS_PER_TILE)],
                    acc.at[pl.ds(row0, ROWS_PER_TILE)])
    plsc.subcore_barrier()

    def gather(p, jj, rv, sem):
        pltpu.make_async_copy(h_hbm.at[src_v.at[p, jj]], rv, sem).start()

    def gather_wait(rv, sem):
        pltpu.make_async_copy(h_hbm.at[src_v.at[0, 0]], rv, sem).wait()

    def scatter(p, jj, rv, sem):
        pltpu.make_async_copy(rv, acc.at[dst_v.at[p, jj]], sem).start(add=True)

    def scatter_wait(rv, sem):
        pltpu.make_async_copy(rv, acc.at[dst_v.at[0, 0]], sem).wait()

    pltpu.sync_copy(src_hbm.at[cid, sid, 0], src_v.at[0])
    pltpu.sync_copy(dst_hbm.at[cid, sid, 0], dst_v.at[0])
    gather(0, 0, r0, g0)
    DT = CH // 2

    def step(t, carry):
        m = t // DT
        p = lax.rem(m, 2)
        jj0 = 2 * lax.rem(t, DT)

        @pl.when(t > 0)
        def _():
            scatter_wait(r1, s1)

        @pl.when(jnp.logical_and(lax.rem(t, DT) == 0, t < (NCH - 1) * DT))
        def _():
            pn = lax.rem(m + 1, 2)
            pltpu.make_async_copy(src_hbm.at[cid, sid, m + 1],
                                  src_v.at[pn], ixs).start()
            pltpu.make_async_copy(dst_hbm.at[cid, sid, m + 1],
                                  dst_v.at[pn], ixd).start()

        gather(p, jj0 + 1, r1, g1)
        gather_wait(r0, g0)
        scatter(p, jj0, r0, s0)

        @pl.when(jnp.logical_and(lax.rem(t, DT) == DT - 1,
                                 t < (NCH - 1) * DT))
        def _():
            pltpu.make_async_copy(src_hbm.at[cid, sid, 0],
                                  src_v.at[0], ixs).wait()
            pltpu.make_async_copy(dst_hbm.at[cid, sid, 0],
                                  dst_v.at[0], ixd).wait()

        jn = 2 * t + 2
        mn = jn // CH
        pn0 = lax.rem(mn, 2)
        jjn = jn - mn * CH
        scatter_wait(r0, s0)

        @pl.when(t < K // 2 - 1)
        def _():
            gather(pn0, jjn, r0, g0)

        gather_wait(r1, g1)
        scatter(p, jj0 + 1, r1, s1)
        return carry

    lax.fori_loop(0, K // 2, step, 0)
    scatter_wait(r1, s1)
    plsc.subcore_barrier()
    pltpu.sync_copy(acc.at[pl.ds(row0, ROWS_PER_TILE)],
                    out_hbm.at[cid, pl.ds(row0, ROWS_PER_TILE)])


_agg = functools.partial(
    pl.kernel,
    out_type=jax.ShapeDtypeStruct((NC, NP, D), jnp.float32),
    mesh=plsc.VectorSubcoreMesh(core_axis_name="c", subcore_axis_name="s"),
    scratch_types=[
        pltpu.VMEM_SHARED((NP, D), jnp.float32),
        pltpu.VMEM((2, CH, EB), jnp.int32),
        pltpu.VMEM((2, CH, EB), jnp.int32),
        pltpu.VMEM((EB, D), jnp.float32),
        pltpu.VMEM((EB, D), jnp.float32),
        pltpu.SemaphoreType.DMA,
        pltpu.SemaphoreType.DMA,
        pltpu.SemaphoreType.DMA,
        pltpu.SemaphoreType.DMA,
        pltpu.SemaphoreType.DMA,
        pltpu.SemaphoreType.DMA,
    ],
)(_agg_body)


def _mlp_body(h_ref, p_ref, w1_ref, b1_ref, w2_ref, b2_ref, o_ref):
    z = h_ref[...] + p_ref[0] + p_ref[1]
    z = jnp.maximum(
        jnp.dot(z, w1_ref[...], preferred_element_type=jnp.float32) + b1_ref[...],
        0.0)
    z = jnp.dot(z, w2_ref[...], preferred_element_type=jnp.float32) + b2_ref[...]
    o_ref[...] = jnp.maximum(z, 0.0)


_mlp = pl.pallas_call(
    _mlp_body,
    grid=(NP // R,),
    in_specs=[
        pl.BlockSpec((R, D), lambda i: (i, 0)),
        pl.BlockSpec((NC, R, D), lambda i: (0, i, 0)),
        pl.BlockSpec((D, D), lambda i: (0, 0)),
        pl.BlockSpec((1, D), lambda i: (0, 0)),
        pl.BlockSpec((D, D), lambda i: (0, 0)),
        pl.BlockSpec((1, D), lambda i: (0, 0)),
    ],
    out_specs=pl.BlockSpec((R, D), lambda i: (i, 0)),
    out_shape=jax.ShapeDtypeStruct((NP, D), jnp.float32),
)


def _mlp_pool_body(h_ref, p_ref, batch_ref, w1_ref, b1_ref, w2_ref, b2_ref,
                   wf1_ref, bf1_ref, wf2_ref, bf2_ref, o_ref, pooled):
    i = pl.program_id(0)
    z = h_ref[...] + p_ref[0] + p_ref[1]
    z = jnp.maximum(
        jnp.dot(z, w1_ref[...], preferred_element_type=jnp.float32) + b1_ref[...],
        0.0)
    z = jnp.dot(z, w2_ref[...], preferred_element_type=jnp.float32) + b2_ref[...]
    h3 = jnp.maximum(z, 0.0)
    seg = batch_ref[0, 0, :]
    m = (lax.broadcasted_iota(jnp.int32, (G, R), 0) == seg[None, :]
         ).astype(jnp.float32)
    part = jnp.dot(m, h3, preferred_element_type=jnp.float32)

    @pl.when(i == 0)
    def _():
        pooled[...] = part

    @pl.when(i > 0)
    def _():
        pooled[...] += part

    @pl.when(i == pl.num_programs(0) - 1)
    def _():
        q = jnp.maximum(
            jnp.dot(pooled[...], wf1_ref[...],
                    preferred_element_type=jnp.float32) + bf1_ref[...],
            0.0)
        o_ref[...] = (jnp.dot(q, wf2_ref[...],
                              preferred_element_type=jnp.float32)
                      + bf2_ref[...])


_mlp_pool = pl.pallas_call(
    _mlp_pool_body,
    grid=(NP // R,),
    in_specs=[
        pl.BlockSpec((R, D), lambda i: (i, 0)),
        pl.BlockSpec((NC, R, D), lambda i: (0, i, 0)),
        pl.BlockSpec((1, 1, R), lambda i: (i, 0, 0)),
        pl.BlockSpec((D, D), lambda i: (0, 0)),
        pl.BlockSpec((1, D), lambda i: (0, 0)),
        pl.BlockSpec((D, D), lambda i: (0, 0)),
        pl.BlockSpec((1, D), lambda i: (0, 0)),
        pl.BlockSpec((D, D), lambda i: (0, 0)),
        pl.BlockSpec((1, D), lambda i: (0, 0)),
        pl.BlockSpec((D, D), lambda i: (0, 0)),
        pl.BlockSpec((1, D), lambda i: (0, 0)),
    ],
    out_specs=pl.BlockSpec((G, D), lambda i: (0, 0)),
    out_shape=jax.ShapeDtypeStruct((G, D), jnp.float32),
    scratch_shapes=[pltpu.VMEM((G, D), jnp.float32)],
)


def kernel(x, edge_index, edge_attr, batch,
           W1_0, b1_0, W2_0, b2_0, W1_1, b1_1, W2_1, b2_1,
           W1_2, b1_2, W2_2, b2_2, Wf1, bf1, Wf2, bf2):
    del edge_attr
    pad = EP - E
    pad_src = (np.arange(pad, dtype=np.int32) * 97) % N
    pad_dst = N + (np.arange(pad, dtype=np.int32) % (NP - N))
    src = jnp.concatenate([edge_index[0].astype(jnp.int32), jnp.asarray(pad_src)])
    dst = jnp.concatenate([edge_index[1].astype(jnp.int32), jnp.asarray(pad_dst)])
    src = src.reshape(NC, NS, NCH, CH, EB)
    dst = dst.reshape(NC, NS, NCH, CH, EB)
    zeros = jnp.zeros((NP, D), jnp.float32)
    h = jnp.concatenate([x, jnp.zeros((NP - N, D), x.dtype)])
    batch_p = jnp.concatenate(
        [batch.astype(jnp.int32),
         jnp.full((NP - N,), G, jnp.int32)]).reshape(NP // R, 1, R)

    parts = _agg(h, src, dst, zeros)
    h = _mlp(h, parts, W1_0, b1_0.reshape(1, D), W2_0, b2_0.reshape(1, D))
    parts = _agg(h, src, dst, zeros)
    h = _mlp(h, parts, W1_1, b1_1.reshape(1, D), W2_1, b2_1.reshape(1, D))
    parts = _agg(h, src, dst, zeros)
    out = _mlp_pool(h, parts, batch_p,
                    W1_2, b1_2.reshape(1, D), W2_2, b2_2.reshape(1, D),
                    Wf1, bf1.reshape(1, D), Wf2, bf2.reshape(1, D))
    return out


# speedup vs baseline: 1.1475x; 1.0187x over previous
import functools

import jax
import jax.numpy as jnp
import numpy as np
from jax import lax
from jax.experimental import pallas as pl
from jax.experimental.pallas import tpu as pltpu
from jax.experimental.pallas import tpu_sc as plsc

N = 10000
E = 320000
D = 128
G = 64
NC, NS = 2, 16
NP = 10240
R = 1000
EB = 128
K = 80
CH = 20
NCH = K // CH
EP = NC * NS * K * EB
ROWS_PER_TILE = NP // NS


def _agg_body(h_hbm, src_hbm, dst_hbm, zeros_hbm, out_hbm,
              acc, src_v, dst_v, r0, r1, g0, g1, s0, s1, ixs, ixd):
    cid = lax.axis_index("c")
    sid = lax.axis_index("s")
    row0 = sid * ROWS_PER_TILE

    def gather(p, jj, rv, sem):
        pltpu.make_async_copy(h_hbm.at[src_v.at[p, jj]], rv, sem).start()

    def gather_wait(rv, sem):
        pltpu.make_async_copy(h_hbm.at[src_v.at[0, 0]], rv, sem).wait()

    def scatter(p, jj, rv, sem):
        pltpu.make_async_copy(rv, acc.at[dst_v.at[p, jj]], sem).start(add=True)

    def scatter_wait(rv, sem):
        pltpu.make_async_copy(rv, acc.at[dst_v.at[0, 0]], sem).wait()

    pltpu.sync_copy(src_hbm.at[cid, sid, 0], src_v.at[0])
    pltpu.sync_copy(dst_hbm.at[cid, sid, 0], dst_v.at[0])
    gather(0, 0, r0, g0)
    pltpu.sync_copy(zeros_hbm.at[pl.ds(row0, ROWS_PER_TILE)],
                    acc.at[pl.ds(row0, ROWS_PER_TILE)])
    plsc.subcore_barrier()
    DT = CH // 2

    def step(t, carry):
        m = t // DT
        p = lax.rem(m, 2)
        jj0 = 2 * lax.rem(t, DT)

        @pl.when(t > 0)
        def _():
            scatter_wait(r1, s1)

        @pl.when(jnp.logical_and(lax.rem(t, DT) == 0, t < (NCH - 1) * DT))
        def _():
            pn = lax.rem(m + 1, 2)
            pltpu.make_async_copy(src_hbm.at[cid, sid, m + 1],
                                  src_v.at[pn], ixs).start()
            pltpu.make_async_copy(dst_hbm.at[cid, sid, m + 1],
                                  dst_v.at[pn], ixd).start()

        gather(p, jj0 + 1, r1, g1)
        gather_wait(r0, g0)
        scatter(p, jj0, r0, s0)

        @pl.when(jnp.logical_and(lax.rem(t, DT) == DT - 1,
                                 t < (NCH - 1) * DT))
        def _():
            pltpu.make_async_copy(src_hbm.at[cid, sid, 0],
                                  src_v.at[0], ixs).wait()
            pltpu.make_async_copy(dst_hbm.at[cid, sid, 0],
                                  dst_v.at[0], ixd).wait()

        jn = 2 * t + 2
        mn = jn // CH
        pn0 = lax.rem(mn, 2)
        jjn = jn - mn * CH
        scatter_wait(r0, s0)

        @pl.when(t < K // 2 - 1)
        def _():
            gather(pn0, jjn, r0, g0)

        gather_wait(r1, g1)
        scatter(p, jj0 + 1, r1, s1)
        return carry

    lax.fori_loop(0, K // 2, step, 0)
    scatter_wait(r1, s1)
    plsc.subcore_barrier()
    pltpu.sync_copy(acc.at[pl.ds(row0, ROWS_PER_TILE)],
                    out_hbm.at[cid, pl.ds(row0, ROWS_PER_TILE)])


_agg = functools.partial(
    pl.kernel,
    out_type=jax.ShapeDtypeStruct((NC, NP, D), jnp.float32),
    mesh=plsc.VectorSubcoreMesh(core_axis_name="c", subcore_axis_name="s"),
    scratch_types=[
        pltpu.VMEM_SHARED((NP, D), jnp.float32),
        pltpu.VMEM((2, CH, EB), jnp.int32),
        pltpu.VMEM((2, CH, EB), jnp.int32),
        pltpu.VMEM((EB, D), jnp.float32),
        pltpu.VMEM((EB, D), jnp.float32),
        pltpu.SemaphoreType.DMA,
        pltpu.SemaphoreType.DMA,
        pltpu.SemaphoreType.DMA,
        pltpu.SemaphoreType.DMA,
        pltpu.SemaphoreType.DMA,
        pltpu.SemaphoreType.DMA,
    ],
)(_agg_body)


def _mlp_body(h_ref, p_ref, w1_ref, b1_ref, w2_ref, b2_ref, o_ref):
    z = h_ref[...] + p_ref[0] + p_ref[1]
    z = jnp.maximum(
        jnp.dot(z, w1_ref[...], preferred_element_type=jnp.float32) + b1_ref[...],
        0.0)
    z = jnp.dot(z, w2_ref[...], preferred_element_type=jnp.float32) + b2_ref[...]
    o_ref[...] = jnp.maximum(z, 0.0)


_mlp = pl.pallas_call(
    _mlp_body,
    grid=(N // R,),
    in_specs=[
        pl.BlockSpec((R, D), lambda i: (i, 0)),
        pl.BlockSpec((NC, R, D), lambda i: (0, i, 0)),
        pl.BlockSpec((D, D), lambda i: (0, 0)),
        pl.BlockSpec((1, D), lambda i: (0, 0)),
        pl.BlockSpec((D, D), lambda i: (0, 0)),
        pl.BlockSpec((1, D), lambda i: (0, 0)),
    ],
    out_specs=pl.BlockSpec((R, D), lambda i: (i, 0)),
    out_shape=jax.ShapeDtypeStruct((N, D), jnp.float32),
)


def _mlp_pool_body(h_ref, p_ref, batch_ref, w1_ref, b1_ref, w2_ref, b2_ref,
                   wf1_ref, bf1_ref, wf2_ref, bf2_ref, o_ref, pooled):
    i = pl.program_id(0)
    z = h_ref[...] + p_ref[0] + p_ref[1]
    z = jnp.maximum(
        jnp.dot(z, w1_ref[...], preferred_element_type=jnp.float32) + b1_ref[...],
        0.0)
    z = jnp.dot(z, w2_ref[...], preferred_element_type=jnp.float32) + b2_ref[...]
    h3 = jnp.maximum(z, 0.0)
    seg = batch_ref[0, 0, :]
    m = (lax.broadcasted_iota(jnp.int32, (G, R), 0) == seg[None, :]
         ).astype(jnp.float32)
    part = jnp.dot(m, h3, preferred_element_type=jnp.float32)

    @pl.when(i == 0)
    def _():
        pooled[...] = part

    @pl.when(i > 0)
    def _():
        pooled[...] += part

    @pl.when(i == pl.num_programs(0) - 1)
    def _():
        q = jnp.maximum(
            jnp.dot(pooled[...], wf1_ref[...],
                    preferred_element_type=jnp.float32) + bf1_ref[...],
            0.0)
        o_ref[...] = (jnp.dot(q, wf2_ref[...],
                              preferred_element_type=jnp.float32)
                      + bf2_ref[...])


_mlp_pool = pl.pallas_call(
    _mlp_pool_body,
    grid=(N // R,),
    in_specs=[
        pl.BlockSpec((R, D), lambda i: (i, 0)),
        pl.BlockSpec((NC, R, D), lambda i: (0, i, 0)),
        pl.BlockSpec((1, 1, R), lambda i: (i, 0, 0)),
        pl.BlockSpec((D, D), lambda i: (0, 0)),
        pl.BlockSpec((1, D), lambda i: (0, 0)),
        pl.BlockSpec((D, D), lambda i: (0, 0)),
        pl.BlockSpec((1, D), lambda i: (0, 0)),
        pl.BlockSpec((D, D), lambda i: (0, 0)),
        pl.BlockSpec((1, D), lambda i: (0, 0)),
        pl.BlockSpec((D, D), lambda i: (0, 0)),
        pl.BlockSpec((1, D), lambda i: (0, 0)),
    ],
    out_specs=pl.BlockSpec((G, D), lambda i: (0, 0)),
    out_shape=jax.ShapeDtypeStruct((G, D), jnp.float32),
    scratch_shapes=[pltpu.VMEM((G, D), jnp.float32)],
)


def kernel(x, edge_index, edge_attr, batch,
           W1_0, b1_0, W2_0, b2_0, W1_1, b1_1, W2_1, b2_1,
           W1_2, b1_2, W2_2, b2_2, Wf1, bf1, Wf2, bf2):
    del edge_attr
    pad = EP - E
    pad_src = (np.arange(pad, dtype=np.int32) * 97) % N
    pad_dst = N + (np.arange(pad, dtype=np.int32) % (NP - N))
    src = jnp.concatenate([edge_index[0].astype(jnp.int32), jnp.asarray(pad_src)])
    dst = jnp.concatenate([edge_index[1].astype(jnp.int32), jnp.asarray(pad_dst)])
    src = src.reshape(NC, NS, NCH, CH, EB)
    dst = dst.reshape(NC, NS, NCH, CH, EB)
    zeros = jnp.zeros((NP, D), jnp.float32)
    h = x
    batch_p = batch.astype(jnp.int32).reshape(N // R, 1, R)

    parts = _agg(h, src, dst, zeros)
    h = _mlp(h, parts, W1_0, b1_0.reshape(1, D), W2_0, b2_0.reshape(1, D))
    parts = _agg(h, src, dst, zeros)
    h = _mlp(h, parts, W1_1, b1_1.reshape(1, D), W2_1, b2_1.reshape(1, D))
    parts = _agg(h, src, dst, zeros)
    out = _mlp_pool(h, parts, batch_p,
                    W1_2, b1_2.reshape(1, D), W2_2, b2_2.reshape(1, D),
                    Wf1, bf1.reshape(1, D), Wf2, bf2.reshape(1, D))
    return out


# speedup vs baseline: 1.1523x; 1.0042x over previous
import functools

import jax
import jax.numpy as jnp
import numpy as np
from jax import lax
from jax.experimental import pallas as pl
from jax.experimental.pallas import tpu as pltpu
from jax.experimental.pallas import tpu_sc as plsc

N = 10000
E = 320000
D = 128
G = 64
NC, NS = 2, 16
NP = 10240
R = 1000
EB = 128
K = 80
CH = 20
NCH = K // CH
EP = NC * NS * K * EB
ROWS_PER_TILE = NP // NS


def _agg_body(h_hbm, src_hbm, dst_hbm, zeros_hbm, out_hbm,
              acc, src_v, dst_v, r0, r1, g0, g1, s0, s1, ixs, ixd):
    cid = lax.axis_index("c")
    sid = lax.axis_index("s")
    row0 = sid * ROWS_PER_TILE

    def gather(p, jj, rv, sem):
        pltpu.make_async_copy(h_hbm.at[src_v.at[p, jj]], rv, sem).start()

    def gather_wait(rv, sem):
        pltpu.make_async_copy(h_hbm.at[src_v.at[0, 0]], rv, sem).wait()

    def scatter(p, jj, rv, sem):
        pltpu.make_async_copy(rv, acc.at[dst_v.at[p, jj]], sem).start(add=True)

    def scatter_wait(rv, sem):
        pltpu.make_async_copy(rv, acc.at[dst_v.at[0, 0]], sem).wait()

    pltpu.sync_copy(src_hbm.at[cid, sid, 0], src_v.at[0])
    pltpu.sync_copy(dst_hbm.at[cid, sid, 0], dst_v.at[0])
    gather(0, 0, r0, g0)
    gather(0, 1, r1, g1)
    pltpu.sync_copy(zeros_hbm.at[pl.ds(row0, ROWS_PER_TILE)],
                    acc.at[pl.ds(row0, ROWS_PER_TILE)])
    plsc.subcore_barrier()
    DT = CH // 2

    def step(t, carry):
        m = t // DT
        p = lax.rem(m, 2)
        jj0 = 2 * lax.rem(t, DT)

        @pl.when(t > 0)
        def _():
            scatter_wait(r1, s1)
            gather(p, jj0 + 1, r1, g1)

        @pl.when(jnp.logical_and(lax.rem(t, DT) == 0, t < (NCH - 1) * DT))
        def _():
            pn = lax.rem(m + 1, 2)
            pltpu.make_async_copy(src_hbm.at[cid, sid, m + 1],
                                  src_v.at[pn], ixs).start()
            pltpu.make_async_copy(dst_hbm.at[cid, sid, m + 1],
                                  dst_v.at[pn], ixd).start()

        gather_wait(r0, g0)
        scatter(p, jj0, r0, s0)

        @pl.when(jnp.logical_and(lax.rem(t, DT) == DT - 1,
                                 t < (NCH - 1) * DT))
        def _():
            pltpu.make_async_copy(src_hbm.at[cid, sid, 0],
                                  src_v.at[0], ixs).wait()
            pltpu.make_async_copy(dst_hbm.at[cid, sid, 0],
                                  dst_v.at[0], ixd).wait()

        jn = 2 * t + 2
        mn = jn // CH
        pn0 = lax.rem(mn, 2)
        jjn = jn - mn * CH
        scatter_wait(r0, s0)

        @pl.when(t < K // 2 - 1)
        def _():
            gather(pn0, jjn, r0, g0)

        gather_wait(r1, g1)
        scatter(p, jj0 + 1, r1, s1)
        return carry

    lax.fori_loop(0, K // 2, step, 0)
    scatter_wait(r1, s1)
    plsc.subcore_barrier()
    pltpu.sync_copy(acc.at[pl.ds(row0, ROWS_PER_TILE)],
                    out_hbm.at[cid, pl.ds(row0, ROWS_PER_TILE)])


_agg = functools.partial(
    pl.kernel,
    out_type=jax.ShapeDtypeStruct((NC, NP, D), jnp.float32),
    mesh=plsc.VectorSubcoreMesh(core_axis_name="c", subcore_axis_name="s"),
    scratch_types=[
        pltpu.VMEM_SHARED((NP, D), jnp.float32),
        pltpu.VMEM((2, CH, EB), jnp.int32),
        pltpu.VMEM((2, CH, EB), jnp.int32),
        pltpu.VMEM((EB, D), jnp.float32),
        pltpu.VMEM((EB, D), jnp.float32),
        pltpu.SemaphoreType.DMA,
        pltpu.SemaphoreType.DMA,
        pltpu.SemaphoreType.DMA,
        pltpu.SemaphoreType.DMA,
        pltpu.SemaphoreType.DMA,
        pltpu.SemaphoreType.DMA,
    ],
)(_agg_body)


def _mlp_body(h_ref, p_ref, w1_ref, b1_ref, w2_ref, b2_ref, o_ref):
    z = h_ref[...] + p_ref[0] + p_ref[1]
    z = jnp.maximum(
        jnp.dot(z, w1_ref[...], preferred_element_type=jnp.float32) + b1_ref[...],
        0.0)
    z = jnp.dot(z, w2_ref[...], preferred_element_type=jnp.float32) + b2_ref[...]
    o_ref[...] = jnp.maximum(z, 0.0)


_mlp = pl.pallas_call(
    _mlp_body,
    grid=(N // R,),
    in_specs=[
        pl.BlockSpec((R, D), lambda i: (i, 0)),
        pl.BlockSpec((NC, R, D), lambda i: (0, i, 0)),
        pl.BlockSpec((D, D), lambda i: (0, 0)),
        pl.BlockSpec((1, D), lambda i: (0, 0)),
        pl.BlockSpec((D, D), lambda i: (0, 0)),
        pl.BlockSpec((1, D), lambda i: (0, 0)),
    ],
    out_specs=pl.BlockSpec((R, D), lambda i: (i, 0)),
    out_shape=jax.ShapeDtypeStruct((N, D), jnp.float32),
)


def _mlp_pool_body(h_ref, p_ref, batch_ref, w1_ref, b1_ref, w2_ref, b2_ref,
                   wf1_ref, bf1_ref, wf2_ref, bf2_ref, o_ref, pooled):
    i = pl.program_id(0)
    z = h_ref[...] + p_ref[0] + p_ref[1]
    z = jnp.maximum(
        jnp.dot(z, w1_ref[...], preferred_element_type=jnp.float32) + b1_ref[...],
        0.0)
    z = jnp.dot(z, w2_ref[...], preferred_element_type=jnp.float32) + b2_ref[...]
    h3 = jnp.maximum(z, 0.0)
    seg = batch_ref[0, 0, :]
    m = (lax.broadcasted_iota(jnp.int32, (G, R), 0) == seg[None, :]
         ).astype(jnp.float32)
    part = jnp.dot(m, h3, preferred_element_type=jnp.float32)

    @pl.when(i == 0)
    def _():
        pooled[...] = part

    @pl.when(i > 0)
    def _():
        pooled[...] += part

    @pl.when(i == pl.num_programs(0) - 1)
    def _():
        q = jnp.maximum(
            jnp.dot(pooled[...], wf1_ref[...],
                    preferred_element_type=jnp.float32) + bf1_ref[...],
            0.0)
        o_ref[...] = (jnp.dot(q, wf2_ref[...],
                              preferred_element_type=jnp.float32)
                      + bf2_ref[...])


_mlp_pool = pl.pallas_call(
    _mlp_pool_body,
    grid=(N // R,),
    in_specs=[
        pl.BlockSpec((R, D), lambda i: (i, 0)),
        pl.BlockSpec((NC, R, D), lambda i: (0, i, 0)),
        pl.BlockSpec((1, 1, R), lambda i: (i, 0, 0)),
        pl.BlockSpec((D, D), lambda i: (0, 0)),
        pl.BlockSpec((1, D), lambda i: (0, 0)),
        pl.BlockSpec((D, D), lambda i: (0, 0)),
        pl.BlockSpec((1, D), lambda i: (0, 0)),
        pl.BlockSpec((D, D), lambda i: (0, 0)),
        pl.BlockSpec((1, D), lambda i: (0, 0)),
        pl.BlockSpec((D, D), lambda i: (0, 0)),
        pl.BlockSpec((1, D), lambda i: (0, 0)),
    ],
    out_specs=pl.BlockSpec((G, D), lambda i: (0, 0)),
    out_shape=jax.ShapeDtypeStruct((G, D), jnp.float32),
    scratch_shapes=[pltpu.VMEM((G, D), jnp.float32)],
)


def kernel(x, edge_index, edge_attr, batch,
           W1_0, b1_0, W2_0, b2_0, W1_1, b1_1, W2_1, b2_1,
           W1_2, b1_2, W2_2, b2_2, Wf1, bf1, Wf2, bf2):
    del edge_attr
    pad = EP - E
    pad_src = (np.arange(pad, dtype=np.int32) * 97) % N
    pad_dst = N + (np.arange(pad, dtype=np.int32) % (NP - N))
    src = jnp.concatenate([edge_index[0].astype(jnp.int32), jnp.asarray(pad_src)])
    dst = jnp.concatenate([edge_index[1].astype(jnp.int32), jnp.asarray(pad_dst)])
    src = src.reshape(NC, NS, NCH, CH, EB)
    dst = dst.reshape(NC, NS, NCH, CH, EB)
    zeros = jnp.zeros((NP, D), jnp.float32)
    h = x
    batch_p = batch.astype(jnp.int32).reshape(N // R, 1, R)

    parts = _agg(h, src, dst, zeros)
    h = _mlp(h, parts, W1_0, b1_0.reshape(1, D), W2_0, b2_0.reshape(1, D))
    parts = _agg(h, src, dst, zeros)
    h = _mlp(h, parts, W1_1, b1_1.reshape(1, D), W2_1, b2_1.reshape(1, D))
    parts = _agg(h, src, dst, zeros)
    out = _mlp_pool(h, parts, batch_p,
                    W1_2, b1_2.reshape(1, D), W2_2, b2_2.reshape(1, D),
                    Wf1, bf1.reshape(1, D), Wf2, bf2.reshape(1, D))
    return out
